# baseline (speedup 1.0000x reference)
import functools
import math

import jax
import jax.numpy as jnp
from jax import lax
from jax.experimental import pallas as pl
from jax.experimental.pallas import tpu as pltpu
from jax.experimental.pallas import tpu_sc as plsc

_EMB = 64
_SCALE = math.sqrt(_EMB)

_NC = 2
_NS = 16
_NW = _NC * _NS

_CHUNK = 512


def _make_gather(n_idx: int, vocab: int, emb: int):
    assert emb % 16 == 0
    assert n_idx % (_NW * 2 * _CHUNK) == 0
    idx_per_w = n_idx // _NW
    slabs = idx_per_w // _CHUNK
    pairs = slabs // 2

    mesh = plsc.VectorSubcoreMesh(core_axis_name="c", subcore_axis_name="s")

    @functools.partial(
        pl.kernel,
        out_type=jax.ShapeDtypeStruct((n_idx, emb), jnp.float32),
        mesh=mesh,
        scratch_types=[
            pltpu.VMEM((slabs, _CHUNK), jnp.int32),
            pltpu.VMEM((_CHUNK, emb), jnp.float32),
            pltpu.VMEM((_CHUNK, emb), jnp.float32),
            pltpu.SemaphoreType.DMA,
            pltpu.SemaphoreType.DMA,
        ],
        compiler_params=pltpu.CompilerParams(use_tc_tiling_on_sc=False),
    )
    def gather_scale(idx_hbm, table_hbm, out_hbm, idx_v, rows0, rows1, s0, s1):
        wid = lax.axis_index("s") * _NC + lax.axis_index("c")
        row0 = wid * idx_per_w

        pltpu.sync_copy(idx_hbm.at[pl.ds(wid * slabs, slabs)], idx_v)

        def fire(slab, buf, sem):
            s = jnp.minimum(slab, slabs - 1)
            pltpu.async_copy(table_hbm.at[idx_v.at[s]], buf, sem)

        def drain(buf, sem):
            pltpu.make_async_copy(table_hbm.at[idx_v.at[0]], buf, sem).wait()

        def scale(buf):
            @plsc.parallel_loop(0, _CHUNK, unroll=4)
            def _(i):
                for k in range(emb // 16):
                    sl = pl.ds(k * 16, 16)
                    buf[i, sl] = buf[i, sl] * _SCALE

        fire(0, rows0, s0)

        def pair_body(gg, carry):
            g = 2 * gg
            out_base = row0 + g * _CHUNK
            fire(g + 1, rows1, s1)
            drain(rows0, s0)
            scale(rows0)
            pltpu.sync_copy(rows0, out_hbm.at[pl.ds(out_base, _CHUNK)])
            fire(g + 2, rows0, s0)
            drain(rows1, s1)
            scale(rows1)
            pltpu.sync_copy(rows1, out_hbm.at[pl.ds(out_base + _CHUNK, _CHUNK)])
            return carry

        lax.fori_loop(0, pairs, pair_body, 0)
        drain(rows0, s0)

    return gather_scale


def kernel(tokens, table):
    b, s = tokens.shape
    vocab, emb = table.shape
    n = b * s
    idx = tokens.reshape(n // _CHUNK, _CHUNK).astype(jnp.int32)
    out = _make_gather(n, vocab, emb)(idx, table)
    return out.reshape(b, s, emb)


# speedup vs baseline: 1.0058x; 1.0058x over previous
import functools
import math

import jax
import jax.numpy as jnp
from jax import lax
from jax.experimental import pallas as pl
from jax.experimental.pallas import tpu as pltpu
from jax.experimental.pallas import tpu_sc as plsc

_NC = 2
_NS = 16
_NW = _NC * _NS

_SLAB = 2


def _make_gather(n_rows: int, seq: int, vocab: int, emb: int):
    assert emb % 16 == 0
    assert n_rows % (_NW * 2 * _SLAB) == 0
    rows_per_w = n_rows // _NW
    slabs = rows_per_w // _SLAB
    pairs = slabs // 2

    mesh = plsc.VectorSubcoreMesh(core_axis_name="c", subcore_axis_name="s")
    scale_f = math.sqrt(emb)

    @functools.partial(
        pl.kernel,
        out_type=jax.ShapeDtypeStruct((n_rows, seq, emb), jnp.float32),
        mesh=mesh,
        scratch_types=[
            pltpu.VMEM((rows_per_w, seq), jnp.int32),
            pltpu.VMEM((_SLAB, seq, emb), jnp.float32),
            pltpu.VMEM((_SLAB, seq, emb), jnp.float32),
            pltpu.SemaphoreType.DMA,
            pltpu.SemaphoreType.DMA,
        ],
        compiler_params=pltpu.CompilerParams(use_tc_tiling_on_sc=False),
    )
    def gather_scale(idx_hbm, table_hbm, out_hbm, idx_v, rows0, rows1, s0, s1):
        wid = lax.axis_index("s") * _NC + lax.axis_index("c")
        row0 = wid * rows_per_w

        pltpu.sync_copy(idx_hbm.at[pl.ds(row0, rows_per_w)], idx_v)

        def fire(slab, buf, sem):
            s = jnp.minimum(slab, slabs - 1)
            for k in range(_SLAB):
                pltpu.async_copy(
                    table_hbm.at[idx_v.at[s * _SLAB + k]], buf.at[k], sem
                )

        def drain(buf, sem):
            for k in range(_SLAB):
                pltpu.make_async_copy(
                    table_hbm.at[idx_v.at[k]], buf.at[k], sem
                ).wait()

        def scale(buf):
            for k in range(_SLAB):
                @plsc.parallel_loop(0, seq, unroll=4)
                def _(i):
                    for j in range(emb // 16):
                        sl = pl.ds(j * 16, 16)
                        buf[k, i, sl] = buf[k, i, sl] * scale_f

        fire(0, rows0, s0)

        def pair_body(gg, carry):
            g = 2 * gg
            out_base = row0 + g * _SLAB
            fire(g + 1, rows1, s1)
            drain(rows0, s0)
            scale(rows0)
            pltpu.sync_copy(rows0, out_hbm.at[pl.ds(out_base, _SLAB)])
            fire(g + 2, rows0, s0)
            drain(rows1, s1)
            scale(rows1)
            pltpu.sync_copy(rows1, out_hbm.at[pl.ds(out_base + _SLAB, _SLAB)])
            return carry

        lax.fori_loop(0, pairs, pair_body, 0)
        drain(rows0, s0)

    return gather_scale


def kernel(tokens, table):
    b, s = tokens.shape
    vocab, emb = table.shape
    return _make_gather(b, s, vocab, emb)(tokens.astype(jnp.int32), table)


# speedup vs baseline: 1.2158x; 1.2088x over previous
import functools
import math

import jax
import jax.numpy as jnp
from jax import lax
from jax.experimental import pallas as pl
from jax.experimental.pallas import tpu as pltpu
from jax.experimental.pallas import tpu_sc as plsc

_NC = 2
_NS = 16
_NW = _NC * _NS

_IDXW = 128


def _make_gather(n_idx_rows: int, vocab: int, emb: int):
    assert emb % 16 == 0
    rows_per_w = n_idx_rows // _NW
    assert rows_per_w % 2 == 0
    pairs = rows_per_w // 2

    mesh = plsc.VectorSubcoreMesh(core_axis_name="c", subcore_axis_name="s")
    scale_f = math.sqrt(emb)

    @functools.partial(
        pl.kernel,
        out_type=jax.ShapeDtypeStruct((n_idx_rows * _IDXW, 2 * emb), jnp.float32),
        mesh=mesh,
        scratch_types=[
            pltpu.VMEM((rows_per_w, _IDXW), jnp.int32),
            pltpu.VMEM((_IDXW, 2 * emb), jnp.float32),
            pltpu.VMEM((_IDXW, 2 * emb), jnp.float32),
            pltpu.SemaphoreType.DMA,
            pltpu.SemaphoreType.DMA,
        ],
        compiler_params=pltpu.CompilerParams(use_tc_tiling_on_sc=True),
    )
    def gather_scale(idx_hbm, table_hbm, out_hbm, idx_v, rows0, rows1, s0, s1):
        wid = lax.axis_index("s") * _NC + lax.axis_index("c")
        row0 = wid * rows_per_w

        pltpu.sync_copy(idx_hbm.at[pl.ds(row0, rows_per_w)], idx_v)

        def fire(slab, buf, sem):
            s = jnp.minimum(slab, rows_per_w - 1)
            pltpu.async_copy(table_hbm.at[idx_v.at[s]], buf, sem)

        def drain(buf, sem):
            pltpu.make_async_copy(table_hbm.at[idx_v.at[0]], buf, sem).wait()

        def scale(buf):
            @plsc.parallel_loop(0, _IDXW, unroll=4)
            def _(i):
                for j in range(emb // 16):
                    sl = pl.ds(j * 16, 16)
                    buf[i, sl] = buf[i, sl] * scale_f

        fire(0, rows0, s0)

        def pair_body(gg, carry):
            g = 2 * gg
            out_base = (row0 + g) * _IDXW
            fire(g + 1, rows1, s1)
            drain(rows0, s0)
            scale(rows0)
            pltpu.sync_copy(rows0, out_hbm.at[pl.ds(out_base, _IDXW)])
            fire(g + 2, rows0, s0)
            drain(rows1, s1)
            scale(rows1)
            pltpu.sync_copy(rows1, out_hbm.at[pl.ds(out_base + _IDXW, _IDXW)])
            return carry

        lax.fori_loop(0, pairs, pair_body, 0)
        drain(rows0, s0)

    return gather_scale


def kernel(tokens, table):
    b, s = tokens.shape
    vocab, emb = table.shape
    n = b * s
    idx = tokens.reshape(n // _IDXW, _IDXW).astype(jnp.int32)
    table128 = jnp.pad(table, ((0, 0), (0, 2 * emb - emb)))
    out = _make_gather(n // _IDXW, vocab, emb)(idx, table128)
    return out.reshape(b, s, 2 * emb)[:, :, :emb]
